# Initial kernel scaffold; baseline (speedup 1.0000x reference)
import jax
import jax.numpy as jnp
from jax.experimental import pallas as pl


def kernel(x, edge_index, W1, b1, W2, b2):
    raise NotImplementedError("write your pallas kernel here")


# speedup vs baseline: 18.2744x; 18.2744x over previous
import functools

import jax
import jax.numpy as jnp
from jax import lax
from jax.experimental import pallas as pl
from jax.experimental.pallas import tpu as pltpu
from jax.experimental.pallas import tpu_sc as plsc

N = 10000
E = 320000
IN_CH, HID_CH, CLS_CH = 128, 256, 64

NC, NS = 2, 16
NW = NC * NS
CH = 128
EPW = E // NW
NCHUNK = -(-EPW // CH)
EPW_PAD = NCHUNK * CH
NACC = 10240
RPT = NACC // NS

_mesh = plsc.VectorSubcoreMesh(core_axis_name="c", subcore_axis_name="s")


@functools.partial(
    pl.kernel,
    out_type=jax.ShapeDtypeStruct((NC, NACC), jnp.float32),
    mesh=_mesh,
    scratch_types=[
        pltpu.VMEM((NCHUNK, CH), jnp.int32),
        pltpu.VMEM((CH,), jnp.float32),
        pltpu.VMEM_SHARED((NACC,), jnp.float32),
    ],
)
def _sc_degree(dst_hbm, zeros_hbm, deg_hbm, idx_d, ones_v, deg_sh):
    c = lax.axis_index("c")
    s = lax.axis_index("s")
    w = s * NC + c
    pltpu.sync_copy(dst_hbm.at[w], idx_d)
    for i in range(CH // 16):
        ones_v[pl.ds(i * 16, 16)] = jnp.ones((16,), jnp.float32)
    r0 = s * RPT
    pltpu.sync_copy(zeros_hbm.at[pl.ds(r0, RPT)], deg_sh.at[pl.ds(r0, RPT)])
    plsc.subcore_barrier()

    def body(j, carry):
        pltpu.sync_copy(ones_v, deg_sh.at[idx_d.at[j]], add=True)
        return carry

    lax.fori_loop(0, NCHUNK, body, 0)
    plsc.subcore_barrier()
    pltpu.sync_copy(deg_sh.at[pl.ds(r0, RPT)], deg_hbm.at[c, pl.ds(r0, RPT)])


def _make_sc_agg(D):
    params = None if D % 128 == 0 else pltpu.CompilerParams(use_tc_tiling_on_sc=False)

    @functools.partial(
        pl.kernel,
        out_type=jax.ShapeDtypeStruct((NC, NACC, D), jnp.float32),
        mesh=_mesh,
        compiler_params=params,
        scratch_types=[
            pltpu.VMEM((NCHUNK, CH), jnp.int32),
            pltpu.VMEM((NCHUNK, CH), jnp.int32),
            pltpu.VMEM((CH, D), jnp.float32),
            pltpu.VMEM_SHARED((NACC, D), jnp.float32),
            pltpu.SemaphoreType.DMA,
        ],
    )
    def _sc_agg(src_hbm, dst_hbm, feat_hbm, zeros_hbm, out_hbm,
                idx_s, idx_d, rows, acc_sh, sem):
        c = lax.axis_index("c")
        s = lax.axis_index("s")
        w = s * NC + c
        pltpu.sync_copy(src_hbm.at[w], idx_s)
        pltpu.sync_copy(dst_hbm.at[w], idx_d)
        r0 = s * RPT
        pltpu.sync_copy(zeros_hbm.at[pl.ds(r0, RPT)], acc_sh.at[pl.ds(r0, RPT)])
        plsc.subcore_barrier()

        def body(j, carry):
            pltpu.async_copy(feat_hbm.at[idx_s.at[j]], rows, sem).wait()
            pltpu.sync_copy(rows, acc_sh.at[idx_d.at[j]], add=True)
            return carry

        lax.fori_loop(0, NCHUNK, body, 0)
        plsc.subcore_barrier()
        pltpu.sync_copy(acc_sh.at[pl.ds(r0, RPT)], out_hbm.at[c, pl.ds(r0, RPT)])

    return _sc_agg


_sc_agg_in = _make_sc_agg(IN_CH)
_sc_agg_cls = _make_sc_agg(CLS_CH)


_BR = 1000


def _tc_prescale_body(dega, degb, x, xp, dinv):
    di = lax.rsqrt(dega[...] + degb[...] + 1.0)
    dinv[...] = di
    xp[...] = x[...] * di


def _tc_prescale(dega, degb, x):
    grid = (N // _BR,)
    return pl.pallas_call(
        _tc_prescale_body,
        grid=grid,
        in_specs=[
            pl.BlockSpec((_BR, 1), lambda i: (i, 0)),
            pl.BlockSpec((_BR, 1), lambda i: (i, 0)),
            pl.BlockSpec((_BR, IN_CH), lambda i: (i, 0)),
        ],
        out_specs=[
            pl.BlockSpec((_BR, IN_CH), lambda i: (i, 0)),
            pl.BlockSpec((_BR, 1), lambda i: (i, 0)),
        ],
        out_shape=[
            jax.ShapeDtypeStruct((N, IN_CH), jnp.float32),
            jax.ShapeDtypeStruct((N, 1), jnp.float32),
        ],
    )(dega, degb, x)


def _tc_mid_body(z1a, z1b, xp, dinv, W1, b1, W2, y2):
    di = dinv[...]
    u = (z1a[...] + z1b[...] + xp[...]) * di
    h = jnp.dot(u, W1[...], preferred_element_type=jnp.float32) + b1[...]
    h = jnp.maximum(h, 0.0)
    g = jnp.dot(h, W2[...], preferred_element_type=jnp.float32)
    y2[...] = g * di


def _tc_mid(z1a, z1b, xp, dinv, W1, b1, W2):
    grid = (N // _BR,)
    return pl.pallas_call(
        _tc_mid_body,
        grid=grid,
        in_specs=[
            pl.BlockSpec((_BR, IN_CH), lambda i: (i, 0)),
            pl.BlockSpec((_BR, IN_CH), lambda i: (i, 0)),
            pl.BlockSpec((_BR, IN_CH), lambda i: (i, 0)),
            pl.BlockSpec((_BR, 1), lambda i: (i, 0)),
            pl.BlockSpec((IN_CH, HID_CH), lambda i: (0, 0)),
            pl.BlockSpec((1, HID_CH), lambda i: (0, 0)),
            pl.BlockSpec((HID_CH, CLS_CH), lambda i: (0, 0)),
        ],
        out_specs=pl.BlockSpec((_BR, CLS_CH), lambda i: (i, 0)),
        out_shape=jax.ShapeDtypeStruct((N, CLS_CH), jnp.float32),
    )(z1a, z1b, xp, dinv, W1, b1, W2)


def _tc_final_body(z2a, z2b, y2, dinv, b2, out):
    out[...] = (z2a[...] + z2b[...] + y2[...]) * dinv[...] + b2[...]


def _tc_final(z2a, z2b, y2, dinv, b2):
    grid = (N // _BR,)
    return pl.pallas_call(
        _tc_final_body,
        grid=grid,
        in_specs=[
            pl.BlockSpec((_BR, CLS_CH), lambda i: (i, 0)),
            pl.BlockSpec((_BR, CLS_CH), lambda i: (i, 0)),
            pl.BlockSpec((_BR, CLS_CH), lambda i: (i, 0)),
            pl.BlockSpec((_BR, 1), lambda i: (i, 0)),
            pl.BlockSpec((1, CLS_CH), lambda i: (0, 0)),
        ],
        out_specs=pl.BlockSpec((_BR, CLS_CH), lambda i: (i, 0)),
        out_shape=jax.ShapeDtypeStruct((N, CLS_CH), jnp.float32),
    )(z2a, z2b, y2, dinv, b2)


def kernel(x, edge_index, W1, b1, W2, b2):
    src = edge_index[0].astype(jnp.int32).reshape(NW, EPW)
    dst = edge_index[1].astype(jnp.int32).reshape(NW, EPW)
    pad = EPW_PAD - EPW
    srcp = jnp.pad(src, ((0, 0), (0, pad))).reshape(NW, NCHUNK, CH)
    dstp = jnp.pad(dst, ((0, 0), (0, pad)), constant_values=N).reshape(NW, NCHUNK, CH)

    zeros1 = jnp.zeros((NACC,), jnp.float32)
    zeros_in = jnp.zeros((NACC, IN_CH), jnp.float32)
    zeros_cls = jnp.zeros((NACC, CLS_CH), jnp.float32)

    deg = _sc_degree(dstp, zeros1)
    dega = deg[0, :N].reshape(N, 1)
    degb = deg[1, :N].reshape(N, 1)

    xp, dinv = _tc_prescale(dega, degb, x)

    z1 = _sc_agg_in(srcp, dstp, xp, zeros_in)
    y2 = _tc_mid(z1[0, :N], z1[1, :N], xp, dinv, W1,
                 b1.reshape(1, HID_CH), W2)

    z2 = _sc_agg_cls(srcp, dstp, y2, zeros_cls)
    out = _tc_final(z2[0, :N], z2[1, :N], y2, dinv, b2.reshape(1, CLS_CH))
    return out
